# baseline (speedup 1.0000x reference)
import functools

import jax
import jax.numpy as jnp
from jax import lax
from jax.experimental import pallas as pl
from jax.experimental.pallas import tpu as pltpu
from jax.experimental.pallas import tpu_sc as plsc

B = 16384
HB = B // 2
PK = 100001
D = 64
TV = 1000
T = 16

NC = 2
NS = 16
NW = NC * NS
BPW = B // NW
PK_CH = 256

_MESH = dict(core_axis_name="c", subcore_axis_name="s",
             num_cores=NC, num_subcores=NS)
_SC_PARAMS = pltpu.CompilerParams(needs_layout_passes=False,
                                  use_tc_tiling_on_sc=False)


def _sc_pool(title, text_flat):
    @functools.partial(
        pl.kernel,
        out_type=jax.ShapeDtypeStruct((B, D), jnp.float32),
        mesh=plsc.VectorSubcoreMesh(**_MESH),
        compiler_params=_SC_PARAMS,
        scratch_types=[
            pltpu.VMEM((BPW, T), jnp.int32),
            pltpu.VMEM((TV * D,), jnp.float32),
            pltpu.VMEM((BPW, D), jnp.float32),
        ],
    )
    def k(title_hbm, text_hbm, text2_hbm, title_v, table_v, tsum_v):
        wid = lax.axis_index("s") * NC + lax.axis_index("c")
        base = wid * BPW

        pltpu.sync_copy(text_hbm, table_v)
        pltpu.sync_copy(title_hbm.at[pl.ds(base, BPW)], title_v)

        zero16 = jnp.zeros((16,), jnp.float32)
        for kk in range(D // 16):
            table_v[pl.ds(kk * 16, 16)] = zero16

        one16 = jnp.full((16,), 1.0, jnp.float32)

        def row_body(r, carry):
            accs = [jnp.zeros((16,), jnp.float32) for _ in range(D // 16)]
            trow = title_v[r, :]
            cnt = plsc.all_reduce_population_count(trow != 0)
            offs = trow * D
            for t in range(T):
                off = offs[t]
                for kk in range(D // 16):
                    g = table_v[pl.ds(off + kk * 16, 16)]
                    accs[kk] = accs[kk] + g
            scale = one16 / jnp.maximum(cnt.astype(jnp.float32), one16)
            for kk in range(D // 16):
                tsum_v[r, pl.ds(kk * 16, 16)] = accs[kk] * scale
            return carry

        lax.fori_loop(0, BPW, row_body, 0)
        pltpu.sync_copy(tsum_v, text2_hbm.at[pl.ds(base, BPW)])

    return k(title, text_flat)


def _sc_pk(item_id, pk_table, combo, price, ts, wmisc_flat):
    @functools.partial(
        pl.kernel,
        out_type=[
            jax.ShapeDtypeStruct((B, D), jnp.float32),
            jax.ShapeDtypeStruct((B, D), jnp.float32),
        ],
        mesh=plsc.VectorSubcoreMesh(**_MESH),
        compiler_params=_SC_PARAMS,
        scratch_types=[
            pltpu.VMEM((BPW,), jnp.int32),
            pltpu.VMEM((BPW,), jnp.int32),
            pltpu.VMEM((BPW,), jnp.float32),
            pltpu.VMEM((BPW,), jnp.float32),
            pltpu.VMEM((130 * D,), jnp.float32),
            pltpu.VMEM((PK_CH, D), jnp.float32),
            pltpu.VMEM((PK_CH, D), jnp.float32),
            pltpu.VMEM((BPW, D), jnp.float32),
            pltpu.SemaphoreType.DMA,
            pltpu.SemaphoreType.DMA,
        ],
    )
    def k(item_hbm, pk_hbm, combo_hbm, price_hbm, ts_hbm, wm_hbm,
          pk2_hbm, misc2_hbm,
          idx_v, combo_v, price_v, ts_v, wm_v, rows_a, rows_b, misc_v,
          sem_a, sem_b):
        wid = lax.axis_index("s") * NC + lax.axis_index("c")
        base = wid * BPW
        pltpu.sync_copy(item_hbm.at[pl.ds(base, BPW)], idx_v)
        bufs = [rows_a, rows_b]
        sems = [sem_a, sem_b]
        cps = [None, None]
        cps[0] = pltpu.async_copy(
            pk_hbm.at[idx_v.at[pl.ds(0, PK_CH)]], rows_a, sem_a)
        pltpu.sync_copy(combo_hbm.at[pl.ds(base, BPW)], combo_v)
        pltpu.sync_copy(price_hbm.at[pl.ds(base, BPW)], price_v)
        pltpu.sync_copy(ts_hbm.at[pl.ds(base, BPW)], ts_v)
        pltpu.sync_copy(wm_hbm, wm_v)

        wp = [wm_v[pl.ds(128 * D + kk * 16, 16)] for kk in range(D // 16)]
        wt = [wm_v[pl.ds(129 * D + kk * 16, 16)] for kk in range(D // 16)]

        def misc_body(r16, carry):
            r = r16 * 16
            cvec = combo_v[pl.ds(r, 16)] * D
            pvec = price_v[pl.ds(r, 16)]
            tvec = ts_v[pl.ds(r, 16)]
            for j in range(16):
                co = cvec[j]
                p = pvec[j]
                t = tvec[j]
                for kk in range(D // 16):
                    acc = (wm_v[pl.ds(co + kk * 16, 16)]
                           + p * wp[kk] + t * wt[kk])
                    misc_v[r + j, pl.ds(kk * 16, 16)] = acc
            return carry

        lax.fori_loop(0, BPW // 16, misc_body, 0)
        pltpu.sync_copy(misc_v, misc2_hbm.at[pl.ds(base, BPW)])

        for c in range(1, BPW // PK_CH):
            cps[c % 2] = pltpu.async_copy(
                pk_hbm.at[idx_v.at[pl.ds(c * PK_CH, PK_CH)]],
                bufs[c % 2], sems[c % 2])
        for c in range(BPW // PK_CH):
            cps[c % 2].wait()
            pltpu.sync_copy(bufs[c % 2],
                            pk2_hbm.at[pl.ds(base + c * PK_CH, PK_CH)])

    return k(item_id, pk_table, combo, price, ts, wmisc_flat)


def _tc_fnn(pk2, text2, misc2, W1pk, W1text, W2, b2):
    BLK = 2048
    grid = (HB // BLK,)
    row_spec = pl.BlockSpec((BLK, 2 * D), lambda i: (i, 0))
    full_spec = pl.BlockSpec((D, D), lambda i: (0, 0))

    def body(pk_ref, text_ref, misc_ref, w1pk_ref, w1t_ref, w2_ref, b2_ref,
             out_ref):
        halves = []
        for h in (0, D):
            x1 = jnp.dot(pk_ref[:, h:h + D], w1pk_ref[...],
                         preferred_element_type=jnp.float32)
            x1 += jnp.dot(text_ref[:, h:h + D], w1t_ref[...],
                          preferred_element_type=jnp.float32)
            x1 += misc_ref[:, h:h + D]
            hrelu = jnp.maximum(x1, 0.0)
            halves.append(jnp.dot(hrelu, w2_ref[...],
                                  preferred_element_type=jnp.float32)
                          + b2_ref[...])
        out_ref[...] = jnp.stack(halves, axis=1).reshape(2 * BLK, D)

    return pl.pallas_call(
        body,
        grid=grid,
        in_specs=[
            row_spec,
            row_spec,
            row_spec,
            full_spec,
            full_spec,
            full_spec,
            pl.BlockSpec((1, D), lambda i: (0, 0)),
        ],
        out_specs=pl.BlockSpec((2 * BLK, D), lambda i: (i, 0)),
        out_shape=jax.ShapeDtypeStruct((B, D), jnp.float32),
    )(pk2, text2, misc2, W1pk, W1text, W2, b2)


def kernel(item_id, category, brand, title, price, created_at,
           pk_table, text_table, W1, b1, W2, b2):
    item_id = item_id.astype(jnp.int32)
    title_i = title.astype(jnp.int32)
    text2 = _sc_pool(title_i, text_table.reshape(-1))
    combo_tab = (W1[D:D + 16][:, None, :] + W1[D + 16:D + 24][None, :, :]
                 + b1[None, None, :]).reshape(128, D)
    wmisc = jnp.concatenate([combo_tab, W1[D + 24 + D:]], axis=0)
    combo = category.astype(jnp.int32) * 8 + brand.astype(jnp.int32)
    pk_rows, misc_rows = _sc_pk(item_id, pk_table, combo,
                                price, created_at, wmisc.reshape(-1))
    pk2 = pk_rows.reshape(HB, 2 * D)
    misc2 = misc_rows.reshape(HB, 2 * D)
    return _tc_fnn(pk2, text2.reshape(HB, 2 * D), misc2,
                   W1[0:D], W1[D + 24:D + 24 + D], W2, b2.reshape(1, D))
